# baseline (speedup 1.0000x reference)
import functools

import jax
import jax.numpy as jnp
from jax import lax
from jax.experimental import pallas as pl
from jax.experimental.pallas import tpu as pltpu
from jax.experimental.pallas import tpu_sc as plsc

NC = 2
NS = 16
NW = NC * NS
BLK = 128
K = 8
KP = 4

_N = 10000
_E = 320000
_NB = -(-_E // (NW * BLK * K)) * K
_EPT = _NB * BLK
_RPT = -(-(_N + 1) // (NS * 8)) * 8
_ACC = _RPT * NS
_R = _RPT
_GRID = NS

_mesh = plsc.VectorSubcoreMesh(
    core_axis_name="c", subcore_axis_name="s", num_cores=NC, num_subcores=NS
)
_sc_params = pltpu.CompilerParams(use_tc_tiling_on_sc=False)


_DEG_NB0 = _NB


def _split(c, s, nb0, nb1):
    nb = jnp.where(c == 0, nb0, nb1)
    base = jnp.where(c == 0, s * nb0, NS * nb0 + s * nb1)
    return nb, base


def _deg_body(dstp_hbm, ones_hbm, zeros_hbm, out_hbm, dst_v, ones_v, acc_sh,
              ssem):
    c = lax.axis_index("c")
    s = lax.axis_index("s")
    base_blk = (s * NC + c) * _NB
    pltpu.sync_copy(zeros_hbm.at[pl.ds(s * _RPT, _RPT)],
                    acc_sh.at[pl.ds(s * _RPT, _RPT)])
    pltpu.sync_copy(ones_hbm, ones_v)
    pltpu.sync_copy(dstp_hbm.at[pl.ds(base_blk, _NB)], dst_v)
    plsc.subcore_barrier()

    def group(g, _):
        sd = [
            pltpu.async_copy(ones_v, acc_sh.at[dst_v.at[g * K + b]], ssem,
                             add=True)
            for b in range(K)
        ]
        for d in sd:
            d.wait()
        return _

    lax.fori_loop(0, _NB // K, group, 0)
    plsc.subcore_barrier()
    pltpu.sync_copy(acc_sh.at[pl.ds(s * _RPT, _RPT)],
                    out_hbm.at[c, pl.ds(s * _RPT, _RPT)])


_deg_kernel = pl.kernel(
    _deg_body,
    out_type=jax.ShapeDtypeStruct((NC, _ACC, 8), jnp.float32),
    mesh=_mesh,
    compiler_params=_sc_params,
    scratch_types=[
        pltpu.VMEM((_NB, BLK), jnp.int32),
        pltpu.VMEM((BLK, 8), jnp.float32),
        pltpu.VMEM_SHARED((_ACC, 8), jnp.float32),
        pltpu.SemaphoreType.DMA,
    ],
)


def _agg_body(kk, g_hbm, srcp_hbm, dstp_hbm, zeros_hbm, out_hbm,
              src_v, dst_v, rows_v, acc_sh, gsem, ssem):
    c = lax.axis_index("c")
    s = lax.axis_index("s")
    tid = s * NC + c
    pltpu.sync_copy(zeros_hbm.at[pl.ds(s * _RPT, _RPT)],
                    acc_sh.at[pl.ds(s * _RPT, _RPT)])
    pltpu.sync_copy(srcp_hbm.at[pl.ds(tid * _NB, _NB)], src_v)
    pltpu.sync_copy(dstp_hbm.at[pl.ds(tid * _NB, _NB)], dst_v)
    plsc.subcore_barrier()

    def group(g, _):
        base = g * kk
        gd = [
            pltpu.async_copy(g_hbm.at[src_v.at[base + b]], rows_v.at[b], gsem)
            for b in range(kk)
        ]
        for d in gd:
            d.wait()
        sd = [
            pltpu.async_copy(rows_v.at[b], acc_sh.at[dst_v.at[base + b]], ssem,
                             add=True)
            for b in range(kk)
        ]
        for d in sd:
            d.wait()
        return _

    lax.fori_loop(0, _NB // kk, group, 0)
    plsc.subcore_barrier()
    pltpu.sync_copy(acc_sh.at[pl.ds(s * _RPT, _RPT)],
                    out_hbm.at[c, pl.ds(s * _RPT, _RPT)])


def _make_agg(h, kk):
    return pl.kernel(
        functools.partial(_agg_body, kk),
        out_type=jax.ShapeDtypeStruct((NC, _ACC, h), jnp.float32),
        mesh=_mesh,
        compiler_params=_sc_params,
        scratch_types=[
            pltpu.VMEM((_NB, BLK), jnp.int32),
            pltpu.VMEM((_NB, BLK), jnp.int32),
            pltpu.VMEM((kk, BLK, h), jnp.float32),
            pltpu.VMEM_SHARED((_ACC, h), jnp.float32),
            pltpu.SemaphoreType.DMA,
            pltpu.SemaphoreType.DMA,
        ],
    )


_agg_kernels = {64: _make_agg(64, 8),
                32: _make_agg(32, 8),
                16: _make_agg(16, 8)}


def _dinv(degp_ref):
    deg = degp_ref[0, :, 0:1] + 1.0
    if NC == 2:
        deg = deg + degp_ref[1, :, 0:1]
    return lax.rsqrt(deg)


def _mm_scale_body(x_ref, w_ref, degp_ref, o_ref):
    o_ref[...] = jnp.dot(
        x_ref[...], w_ref[...], preferred_element_type=jnp.float32
    ) * _dinv(degp_ref)


def _mid_body(aggp_ref, g_ref, degp_ref, b_ref, w_ref, o_ref):
    dinv = _dinv(degp_ref)
    agg = aggp_ref[0] + aggp_ref[1] if NC == 2 else aggp_ref[0]
    z = dinv * (agg + g_ref[...]) + b_ref[...]
    h = jnp.maximum(z, 0.0)
    o_ref[...] = jnp.dot(
        h, w_ref[...], preferred_element_type=jnp.float32
    ) * dinv


def _final_body(aggp_ref, g_ref, degp_ref, b_ref, o_ref):
    dinv = _dinv(degp_ref)
    agg = aggp_ref[0] + aggp_ref[1] if NC == 2 else aggp_ref[0]
    z = dinv * (agg + g_ref[...]) + b_ref[...]
    m = jnp.max(z, axis=1, keepdims=True)
    lse = m + jnp.log(jnp.sum(jnp.exp(z - m), axis=1, keepdims=True))
    o_ref[...] = z - lse


def _mm_scale(x, w, degp):
    f_in, h = w.shape
    return pl.pallas_call(
        _mm_scale_body,
        grid=(_GRID,),
        in_specs=[
            pl.BlockSpec((_R, f_in), lambda i: (i, 0)),
            pl.BlockSpec((f_in, h), lambda i: (0, 0)),
            pl.BlockSpec((NC, _R, 8), lambda i: (0, i, 0)),
        ],
        out_specs=pl.BlockSpec((_R, h), lambda i: (i, 0)),
        out_shape=jax.ShapeDtypeStruct((_ACC, h), jnp.float32),
    )(x, w, degp)


def _mid(aggp, g, degp, b, w):
    h, h2 = w.shape
    return pl.pallas_call(
        _mid_body,
        grid=(_GRID,),
        in_specs=[
            pl.BlockSpec((NC, _R, h), lambda i: (0, i, 0)),
            pl.BlockSpec((_R, h), lambda i: (i, 0)),
            pl.BlockSpec((NC, _R, 8), lambda i: (0, i, 0)),
            pl.BlockSpec((1, h), lambda i: (0, 0)),
            pl.BlockSpec((h, h2), lambda i: (0, 0)),
        ],
        out_specs=pl.BlockSpec((_R, h2), lambda i: (i, 0)),
        out_shape=jax.ShapeDtypeStruct((_ACC, h2), jnp.float32),
    )(aggp, g, degp, b.reshape(1, h), w)


def _final(aggp, g, degp, b):
    c = g.shape[1]
    return pl.pallas_call(
        _final_body,
        grid=(_GRID,),
        in_specs=[
            pl.BlockSpec((NC, _R, c), lambda i: (0, i, 0)),
            pl.BlockSpec((_R, c), lambda i: (i, 0)),
            pl.BlockSpec((NC, _R, 8), lambda i: (0, i, 0)),
            pl.BlockSpec((1, c), lambda i: (0, 0)),
        ],
        out_specs=pl.BlockSpec((_R, c), lambda i: (i, 0)),
        out_shape=jax.ShapeDtypeStruct((_N, c), jnp.float32),
    )(aggp, g, degp, b.reshape(1, c))


@jax.jit
def _gcn(x, edge_index, W1, b1, W2, b2, W3, b3):
    n = x.shape[0]
    e = edge_index.shape[1]
    pad = NW * _EPT - e
    srcp = jnp.concatenate(
        [edge_index[0], jnp.zeros((pad,), jnp.int32)]).reshape(NW * _NB, BLK)
    dstp = jnp.concatenate(
        [edge_index[1], jnp.full((pad,), n, jnp.int32)]).reshape(NW * _NB, BLK)

    degp = _deg_kernel(dstp, jnp.ones((BLK, 8), jnp.float32),
                       jnp.zeros((_ACC, 8), jnp.float32))

    g1 = _mm_scale(x, W1, degp)
    a1 = _agg_kernels[64](g1, srcp, dstp, jnp.zeros((_ACC, 64), jnp.float32))
    g2 = _mid(a1, g1, degp, b1, W2)
    a2 = _agg_kernels[32](g2, srcp, dstp, jnp.zeros((_ACC, 32), jnp.float32))
    g3 = _mid(a2, g2, degp, b2, W3)
    a3 = _agg_kernels[16](g3, srcp, dstp, jnp.zeros((_ACC, 16), jnp.float32))
    return _final(a3, g3, degp, b3)


def kernel(x, edge_index, W1, b1, W2, b2, W3, b3):
    return _gcn(x, edge_index, W1, b1, W2, b2, W3, b3)


# speedup vs baseline: 1.1742x; 1.1742x over previous
import functools

import jax
import jax.numpy as jnp
from jax import lax
from jax.experimental import pallas as pl
from jax.experimental.pallas import tpu as pltpu
from jax.experimental.pallas import tpu_sc as plsc

NC = 2
NS = 16
NW = NC * NS
BLK = 128
K = 8
KP = 4

_N = 10000
_E = 320000
_NB = -(-_E // (NW * BLK * K)) * K
_EPT = _NB * BLK
_RPT = -(-(_N + 1) // (NS * 8)) * 8
_ACC = _RPT * NS
_R = 1000
_GRID = _N // _R

_mesh = plsc.VectorSubcoreMesh(
    core_axis_name="c", subcore_axis_name="s", num_cores=NC, num_subcores=NS
)
_sc_params = pltpu.CompilerParams(use_tc_tiling_on_sc=False)


_DEG_NB0 = _NB


def _split(c, s, nb0, nb1):
    nb = jnp.where(c == 0, nb0, nb1)
    base = jnp.where(c == 0, s * nb0, NS * nb0 + s * nb1)
    return nb, base


def _deg_body(dstp_hbm, ones_hbm, zeros_hbm, out_hbm, dst_v, ones_v, acc_sh,
              ssem):
    c = lax.axis_index("c")
    s = lax.axis_index("s")
    base_blk = (s * NC + c) * _NB
    pltpu.sync_copy(zeros_hbm.at[pl.ds(s * _RPT, _RPT)],
                    acc_sh.at[pl.ds(s * _RPT, _RPT)])
    pltpu.sync_copy(ones_hbm, ones_v)
    pltpu.sync_copy(dstp_hbm.at[pl.ds(base_blk, _NB)], dst_v)
    plsc.subcore_barrier()

    def group(g, _):
        sd = [
            pltpu.async_copy(ones_v, acc_sh.at[dst_v.at[g * K + b]], ssem,
                             add=True)
            for b in range(K)
        ]
        for d in sd:
            d.wait()
        return _

    lax.fori_loop(0, _NB // K, group, 0)
    plsc.subcore_barrier()
    pltpu.sync_copy(acc_sh.at[pl.ds(s * _RPT, _RPT)],
                    out_hbm.at[c, pl.ds(s * _RPT, _RPT)])


_deg_kernel = pl.kernel(
    _deg_body,
    out_type=jax.ShapeDtypeStruct((NC, _ACC, 8), jnp.float32),
    mesh=_mesh,
    compiler_params=_sc_params,
    scratch_types=[
        pltpu.VMEM((_NB, BLK), jnp.int32),
        pltpu.VMEM((BLK, 8), jnp.float32),
        pltpu.VMEM_SHARED((_ACC, 8), jnp.float32),
        pltpu.SemaphoreType.DMA,
    ],
)


def _agg_body(kk, g_hbm, srcp_hbm, dstp_hbm, zeros_hbm, out_hbm,
              src_v, dst_v, rows_v, acc_sh, gsem, ssem):
    c = lax.axis_index("c")
    s = lax.axis_index("s")
    tid = s * NC + c
    pltpu.sync_copy(zeros_hbm.at[pl.ds(s * _RPT, _RPT)],
                    acc_sh.at[pl.ds(s * _RPT, _RPT)])
    pltpu.sync_copy(srcp_hbm.at[pl.ds(tid * _NB, _NB)], src_v)
    pltpu.sync_copy(dstp_hbm.at[pl.ds(tid * _NB, _NB)], dst_v)
    plsc.subcore_barrier()

    def group(g, _):
        base = g * kk
        gd = [
            pltpu.async_copy(g_hbm.at[src_v.at[base + b]], rows_v.at[b], gsem)
            for b in range(kk)
        ]
        for d in gd:
            d.wait()
        sd = [
            pltpu.async_copy(rows_v.at[b], acc_sh.at[dst_v.at[base + b]], ssem,
                             add=True)
            for b in range(kk)
        ]
        for d in sd:
            d.wait()
        return _

    lax.fori_loop(0, _NB // kk, group, 0)
    plsc.subcore_barrier()
    pltpu.sync_copy(acc_sh.at[pl.ds(s * _RPT, _RPT)],
                    out_hbm.at[c, pl.ds(s * _RPT, _RPT)])


def _make_agg(h, kk):
    return pl.kernel(
        functools.partial(_agg_body, kk),
        out_type=jax.ShapeDtypeStruct((NC, _ACC, h), jnp.float32),
        mesh=_mesh,
        compiler_params=_sc_params,
        scratch_types=[
            pltpu.VMEM((_NB, BLK), jnp.int32),
            pltpu.VMEM((_NB, BLK), jnp.int32),
            pltpu.VMEM((kk, BLK, h), jnp.float32),
            pltpu.VMEM_SHARED((_ACC, h), jnp.float32),
            pltpu.SemaphoreType.DMA,
            pltpu.SemaphoreType.DMA,
        ],
    )


_agg_kernels = {64: _make_agg(64, 8),
                32: _make_agg(32, 8),
                16: _make_agg(16, 8)}


def _dinv(degp_ref):
    deg = degp_ref[0, :, 0:1] + 1.0
    if NC == 2:
        deg = deg + degp_ref[1, :, 0:1]
    return lax.rsqrt(deg)


def _mm_scale_body(x_ref, w_ref, degp_ref, o_ref):
    o_ref[...] = jnp.dot(
        x_ref[...], w_ref[...], preferred_element_type=jnp.float32
    ) * _dinv(degp_ref)


def _mid_body(aggp_ref, g_ref, degp_ref, b_ref, w_ref, o_ref):
    dinv = _dinv(degp_ref)
    agg = aggp_ref[0] + aggp_ref[1] if NC == 2 else aggp_ref[0]
    z = dinv * (agg + g_ref[...]) + b_ref[...]
    h = jnp.maximum(z, 0.0)
    o_ref[...] = jnp.dot(
        h, w_ref[...], preferred_element_type=jnp.float32
    ) * dinv


def _final_body(aggp_ref, g_ref, degp_ref, b_ref, o_ref):
    dinv = _dinv(degp_ref)
    agg = aggp_ref[0] + aggp_ref[1] if NC == 2 else aggp_ref[0]
    z = dinv * (agg + g_ref[...]) + b_ref[...]
    m = jnp.max(z, axis=1, keepdims=True)
    lse = m + jnp.log(jnp.sum(jnp.exp(z - m), axis=1, keepdims=True))
    o_ref[...] = z - lse


def _mm_scale(x, w, degp):
    f_in, h = w.shape
    return pl.pallas_call(
        _mm_scale_body,
        grid=(_GRID,),
        in_specs=[
            pl.BlockSpec((_R, f_in), lambda i: (i, 0)),
            pl.BlockSpec((f_in, h), lambda i: (0, 0)),
            pl.BlockSpec((NC, _R, 8), lambda i: (0, i, 0)),
        ],
        out_specs=pl.BlockSpec((_R, h), lambda i: (i, 0)),
        out_shape=jax.ShapeDtypeStruct((_N, h), jnp.float32),
    )(x, w, degp)


def _mid(aggp, g, degp, b, w):
    h, h2 = w.shape
    return pl.pallas_call(
        _mid_body,
        grid=(_GRID,),
        in_specs=[
            pl.BlockSpec((NC, _R, h), lambda i: (0, i, 0)),
            pl.BlockSpec((_R, h), lambda i: (i, 0)),
            pl.BlockSpec((NC, _R, 8), lambda i: (0, i, 0)),
            pl.BlockSpec((1, h), lambda i: (0, 0)),
            pl.BlockSpec((h, h2), lambda i: (0, 0)),
        ],
        out_specs=pl.BlockSpec((_R, h2), lambda i: (i, 0)),
        out_shape=jax.ShapeDtypeStruct((_N, h2), jnp.float32),
    )(aggp, g, degp, b.reshape(1, h), w)


def _final(aggp, g, degp, b):
    c = g.shape[1]
    return pl.pallas_call(
        _final_body,
        grid=(_GRID,),
        in_specs=[
            pl.BlockSpec((NC, _R, c), lambda i: (0, i, 0)),
            pl.BlockSpec((_R, c), lambda i: (i, 0)),
            pl.BlockSpec((NC, _R, 8), lambda i: (0, i, 0)),
            pl.BlockSpec((1, c), lambda i: (0, 0)),
        ],
        out_specs=pl.BlockSpec((_R, c), lambda i: (i, 0)),
        out_shape=jax.ShapeDtypeStruct((_N, c), jnp.float32),
    )(aggp, g, degp, b.reshape(1, c))


@jax.jit
def _gcn(x, edge_index, W1, b1, W2, b2, W3, b3):
    n = x.shape[0]
    e = edge_index.shape[1]
    pad = NW * _EPT - e
    srcp = jnp.concatenate(
        [edge_index[0], jnp.zeros((pad,), jnp.int32)]).reshape(NW * _NB, BLK)
    dstp = jnp.concatenate(
        [edge_index[1], jnp.full((pad,), n, jnp.int32)]).reshape(NW * _NB, BLK)

    degp = _deg_kernel(dstp, jnp.ones((BLK, 8), jnp.float32),
                       jnp.zeros((_ACC, 8), jnp.float32))

    g1 = _mm_scale(x, W1, degp)
    a1 = _agg_kernels[64](g1, srcp, dstp, jnp.zeros((_ACC, 64), jnp.float32))
    g2 = _mid(a1, g1, degp, b1, W2)
    a2 = _agg_kernels[32](g2, srcp, dstp, jnp.zeros((_ACC, 32), jnp.float32))
    g3 = _mid(a2, g2, degp, b2, W3)
    a3 = _agg_kernels[16](g3, srcp, dstp, jnp.zeros((_ACC, 16), jnp.float32))
    return _final(a3, g3, degp, b3)


def kernel(x, edge_index, W1, b1, W2, b2, W3, b3):
    return _gcn(x, edge_index, W1, b1, W2, b2, W3, b3)


# speedup vs baseline: 1.1918x; 1.0150x over previous
import functools

import jax
import jax.numpy as jnp
from jax import lax
from jax.experimental import pallas as pl
from jax.experimental.pallas import tpu as pltpu
from jax.experimental.pallas import tpu_sc as plsc

NC = 2
NS = 16
NW = NC * NS
BLK = 128
K = 8
KP = 4

_N = 10000
_E = 320000
_NB = -(-_E // (NW * BLK * K)) * K
_EPT = _NB * BLK
_RPT = -(-(_N + 1) // (NS * 8)) * 8
_ACC = _RPT * NS
_R = 2000
_GRID = _N // _R

_mesh = plsc.VectorSubcoreMesh(
    core_axis_name="c", subcore_axis_name="s", num_cores=NC, num_subcores=NS
)
_sc_params = pltpu.CompilerParams(use_tc_tiling_on_sc=False)


_DEG_NB0 = _NB


def _split(c, s, nb0, nb1):
    nb = jnp.where(c == 0, nb0, nb1)
    base = jnp.where(c == 0, s * nb0, NS * nb0 + s * nb1)
    return nb, base


def _deg_body(dstp_hbm, ones_hbm, zeros_hbm, out_hbm, dst_v, ones_v, acc_sh,
              ssem):
    c = lax.axis_index("c")
    s = lax.axis_index("s")
    base_blk = (s * NC + c) * _NB
    pltpu.sync_copy(zeros_hbm.at[pl.ds(s * _RPT, _RPT)],
                    acc_sh.at[pl.ds(s * _RPT, _RPT)])
    pltpu.sync_copy(ones_hbm, ones_v)
    pltpu.sync_copy(dstp_hbm.at[pl.ds(base_blk, _NB)], dst_v)
    plsc.subcore_barrier()

    def group(g, _):
        sd = [
            pltpu.async_copy(ones_v, acc_sh.at[dst_v.at[g * K + b]], ssem,
                             add=True)
            for b in range(K)
        ]
        for d in sd:
            d.wait()
        return _

    lax.fori_loop(0, _NB // K, group, 0)
    plsc.subcore_barrier()
    pltpu.sync_copy(acc_sh.at[pl.ds(s * _RPT, _RPT)],
                    out_hbm.at[c, pl.ds(s * _RPT, _RPT)])


_deg_kernel = pl.kernel(
    _deg_body,
    out_type=jax.ShapeDtypeStruct((NC, _ACC, 8), jnp.float32),
    mesh=_mesh,
    compiler_params=_sc_params,
    scratch_types=[
        pltpu.VMEM((_NB, BLK), jnp.int32),
        pltpu.VMEM((BLK, 8), jnp.float32),
        pltpu.VMEM_SHARED((_ACC, 8), jnp.float32),
        pltpu.SemaphoreType.DMA,
    ],
)


def _agg_body(kk, g_hbm, srcp_hbm, dstp_hbm, zeros_hbm, out_hbm,
              src_v, dst_v, rows_v, acc_sh, gsem, ssem):
    c = lax.axis_index("c")
    s = lax.axis_index("s")
    tid = s * NC + c
    pltpu.sync_copy(zeros_hbm.at[pl.ds(s * _RPT, _RPT)],
                    acc_sh.at[pl.ds(s * _RPT, _RPT)])
    pltpu.sync_copy(srcp_hbm.at[pl.ds(tid * _NB, _NB)], src_v)
    pltpu.sync_copy(dstp_hbm.at[pl.ds(tid * _NB, _NB)], dst_v)
    plsc.subcore_barrier()

    def group(g, _):
        base = g * kk
        gd = [
            pltpu.async_copy(g_hbm.at[src_v.at[base + b]], rows_v.at[b], gsem)
            for b in range(kk)
        ]
        for d in gd:
            d.wait()
        sd = [
            pltpu.async_copy(rows_v.at[b], acc_sh.at[dst_v.at[base + b]], ssem,
                             add=True)
            for b in range(kk)
        ]
        for d in sd:
            d.wait()
        return _

    lax.fori_loop(0, _NB // kk, group, 0)
    plsc.subcore_barrier()
    pltpu.sync_copy(acc_sh.at[pl.ds(s * _RPT, _RPT)],
                    out_hbm.at[c, pl.ds(s * _RPT, _RPT)])


def _make_agg(h, kk):
    return pl.kernel(
        functools.partial(_agg_body, kk),
        out_type=jax.ShapeDtypeStruct((NC, _ACC, h), jnp.float32),
        mesh=_mesh,
        compiler_params=_sc_params,
        scratch_types=[
            pltpu.VMEM((_NB, BLK), jnp.int32),
            pltpu.VMEM((_NB, BLK), jnp.int32),
            pltpu.VMEM((kk, BLK, h), jnp.float32),
            pltpu.VMEM_SHARED((_ACC, h), jnp.float32),
            pltpu.SemaphoreType.DMA,
            pltpu.SemaphoreType.DMA,
        ],
    )


_agg_kernels = {64: _make_agg(64, 8),
                32: _make_agg(32, 8),
                16: _make_agg(16, 8)}


def _dinv(degp_ref):
    deg = degp_ref[0, :, 0:1] + 1.0
    if NC == 2:
        deg = deg + degp_ref[1, :, 0:1]
    return lax.rsqrt(deg)


def _mm_scale_body(x_ref, w_ref, degp_ref, o_ref):
    o_ref[...] = jnp.dot(
        x_ref[...], w_ref[...], preferred_element_type=jnp.float32
    ) * _dinv(degp_ref)


def _mid_body(aggp_ref, g_ref, degp_ref, b_ref, w_ref, o_ref):
    dinv = _dinv(degp_ref)
    agg = aggp_ref[0] + aggp_ref[1] if NC == 2 else aggp_ref[0]
    z = dinv * (agg + g_ref[...]) + b_ref[...]
    h = jnp.maximum(z, 0.0)
    o_ref[...] = jnp.dot(
        h, w_ref[...], preferred_element_type=jnp.float32
    ) * dinv


def _final_body(aggp_ref, g_ref, degp_ref, b_ref, o_ref):
    dinv = _dinv(degp_ref)
    agg = aggp_ref[0] + aggp_ref[1] if NC == 2 else aggp_ref[0]
    z = dinv * (agg + g_ref[...]) + b_ref[...]
    m = jnp.max(z, axis=1, keepdims=True)
    lse = m + jnp.log(jnp.sum(jnp.exp(z - m), axis=1, keepdims=True))
    o_ref[...] = z - lse


def _mm_scale(x, w, degp):
    f_in, h = w.shape
    return pl.pallas_call(
        _mm_scale_body,
        grid=(_GRID,),
        in_specs=[
            pl.BlockSpec((_R, f_in), lambda i: (i, 0)),
            pl.BlockSpec((f_in, h), lambda i: (0, 0)),
            pl.BlockSpec((NC, _R, 8), lambda i: (0, i, 0)),
        ],
        out_specs=pl.BlockSpec((_R, h), lambda i: (i, 0)),
        out_shape=jax.ShapeDtypeStruct((_N, h), jnp.float32),
    )(x, w, degp)


def _mid(aggp, g, degp, b, w):
    h, h2 = w.shape
    return pl.pallas_call(
        _mid_body,
        grid=(_GRID,),
        in_specs=[
            pl.BlockSpec((NC, _R, h), lambda i: (0, i, 0)),
            pl.BlockSpec((_R, h), lambda i: (i, 0)),
            pl.BlockSpec((NC, _R, 8), lambda i: (0, i, 0)),
            pl.BlockSpec((1, h), lambda i: (0, 0)),
            pl.BlockSpec((h, h2), lambda i: (0, 0)),
        ],
        out_specs=pl.BlockSpec((_R, h2), lambda i: (i, 0)),
        out_shape=jax.ShapeDtypeStruct((_N, h2), jnp.float32),
    )(aggp, g, degp, b.reshape(1, h), w)


def _final(aggp, g, degp, b):
    c = g.shape[1]
    return pl.pallas_call(
        _final_body,
        grid=(_GRID,),
        in_specs=[
            pl.BlockSpec((NC, _R, c), lambda i: (0, i, 0)),
            pl.BlockSpec((_R, c), lambda i: (i, 0)),
            pl.BlockSpec((NC, _R, 8), lambda i: (0, i, 0)),
            pl.BlockSpec((1, c), lambda i: (0, 0)),
        ],
        out_specs=pl.BlockSpec((_R, c), lambda i: (i, 0)),
        out_shape=jax.ShapeDtypeStruct((_N, c), jnp.float32),
    )(aggp, g, degp, b.reshape(1, c))


@jax.jit
def _gcn(x, edge_index, W1, b1, W2, b2, W3, b3):
    n = x.shape[0]
    e = edge_index.shape[1]
    pad = NW * _EPT - e
    srcp = jnp.concatenate(
        [edge_index[0], jnp.zeros((pad,), jnp.int32)]).reshape(NW * _NB, BLK)
    dstp = jnp.concatenate(
        [edge_index[1], jnp.full((pad,), n, jnp.int32)]).reshape(NW * _NB, BLK)

    degp = _deg_kernel(dstp, jnp.ones((BLK, 8), jnp.float32),
                       jnp.zeros((_ACC, 8), jnp.float32))

    g1 = _mm_scale(x, W1, degp)
    a1 = _agg_kernels[64](g1, srcp, dstp, jnp.zeros((_ACC, 64), jnp.float32))
    g2 = _mid(a1, g1, degp, b1, W2)
    a2 = _agg_kernels[32](g2, srcp, dstp, jnp.zeros((_ACC, 32), jnp.float32))
    g3 = _mid(a2, g2, degp, b2, W3)
    a3 = _agg_kernels[16](g3, srcp, dstp, jnp.zeros((_ACC, 16), jnp.float32))
    return _final(a3, g3, degp, b3)


def kernel(x, edge_index, W1, b1, W2, b2, W3, b3):
    return _gcn(x, edge_index, W1, b1, W2, b2, W3, b3)
